# baseline (speedup 1.0000x reference)
import functools

import jax
import jax.numpy as jnp
from jax import lax
from jax.experimental import pallas as pl
from jax.experimental.pallas import tpu as pltpu
from jax.experimental.pallas import tpu_sc as plsc


def _precompute_body(fma_ref, fmb_ref, wz_ref, wr_ref, wh_ref, bz_ref, bh_ref,
                     fz_ref, fr_ref, fh_ref, h1_ref, *, block_rows, h_size):
    fma = fma_ref[...]
    fmb = fmb_ref[...]

    def packed(w_ref):
        w = w_ref[...]
        a = jnp.dot(fma, w, preferred_element_type=jnp.float32)
        b = jnp.dot(fmb, w, preferred_element_type=jnp.float32)
        return jnp.concatenate([a, b], axis=1)

    fz = packed(wz_ref)
    fr = packed(wr_ref)
    fh = packed(wh_ref)
    fz_ref[...] = fz.astype(fz_ref.dtype)
    fr_ref[...] = fr.astype(fr_ref.dtype)
    fh_ref[...] = fh.astype(fh_ref.dtype)
    z1 = jax.nn.sigmoid(fz + bz_ref[...])
    p1 = jnp.tanh(fh + bh_ref[...])
    h1 = z1 * p1
    row = lax.broadcasted_iota(jnp.int32, h1.shape, 0) + pl.program_id(0) * block_rows
    lane = lax.broadcasted_iota(jnp.int32, h1.shape, 1)
    h1_ref[...] = jnp.where((row == 0) & (lane < h_size), 0.0, h1).astype(h1_ref.dtype)


def _dense_body(hn_ref, fz_ref, fr_ref, fh_ref, uz_ref, ur_ref, uh_ref,
                bz_ref, urb_ref, bh_ref, *out_refs, n_nei, block_rows, h_size):
    fr = fr_ref[...].astype(jnp.float32)
    ur = ur_ref[...]
    urb = urb_ref[...]
    sum_h = jnp.zeros_like(fr)
    sum_g = jnp.zeros_like(fr)
    for n in range(n_nei):
        hn = hn_ref[n].astype(jnp.float32)
        r = jax.nn.sigmoid(
            fr + jnp.dot(hn, ur, preferred_element_type=jnp.float32) + urb)
        sum_h = sum_h + hn
        sum_g = sum_g + r * hn
    z = jax.nn.sigmoid(
        fz_ref[...].astype(jnp.float32)
        + jnp.dot(sum_h, uz_ref[...], preferred_element_type=jnp.float32)
        + bz_ref[...])
    pre = jnp.tanh(
        fh_ref[...].astype(jnp.float32)
        + jnp.dot(sum_g, uh_ref[...], preferred_element_type=jnp.float32)
        + bh_ref[...])
    nh = (1.0 - z) * sum_h + z * pre
    row = lax.broadcasted_iota(jnp.int32, nh.shape, 0) + pl.program_id(0) * block_rows
    lane = lax.broadcasted_iota(jnp.int32, nh.shape, 1)
    nh = jnp.where((row == 0) & (lane < h_size), 0.0, nh)
    if len(out_refs) == 1:
        out_refs[0][...] = nh.astype(out_refs[0].dtype)
    else:
        out_refs[0][...] = nh[:, :h_size].astype(out_refs[0].dtype)
        out_refs[1][...] = nh[:, h_size:].astype(out_refs[1].dtype)


def _make_gather(e_total, h_size, n_nei, dtype):
    info = plsc.get_sparse_core_info()
    nc, ns = info.num_cores, info.num_subcores
    nw = nc * ns
    total = e_total * n_nei
    per_w = total // nw
    assert per_w * nw == total and per_w % 8 == 0
    k = 400
    n_chunks = per_w // k
    assert n_chunks * k == per_w and n_chunks % 2 == 0 and k % 8 == 0
    n_pairs = n_chunks // 2

    mesh = plsc.VectorSubcoreMesh(core_axis_name="c", subcore_axis_name="s")

    @functools.partial(
        pl.kernel,
        mesh=mesh,
        out_type=jax.ShapeDtypeStruct((total, h_size), dtype),
        scratch_types=[
            pltpu.VMEM((per_w,), jnp.int32),
            pltpu.VMEM((2, k, h_size), dtype),
            pltpu.SemaphoreType.DMA,
            pltpu.SemaphoreType.DMA,
            pltpu.SemaphoreType.DMA,
            pltpu.SemaphoreType.DMA,
        ],
        compiler_params=pltpu.CompilerParams(use_tc_tiling_on_sc=False),
    )
    def gather(h_hbm, idx_hbm, out_hbm, idx_v, rows_v, sg0, sg1, sw0, sw1):
        wid = lax.axis_index("s") * nc + lax.axis_index("c")
        base = wid * per_w
        sg = (sg0, sg1)
        sw = (sw0, sw1)

        pltpu.sync_copy(idx_hbm.at[pl.ds(base, per_w)], idx_v)

        def pair(g, carry):
            for b in range(2):
                i = g * 2 + b
                @pl.when(g > 0)
                def _():
                    pltpu.make_async_copy(
                        rows_v.at[b], out_hbm.at[pl.ds(base, k)], sw[b]).wait()

                pltpu.async_copy(
                    h_hbm.at[idx_v.at[pl.ds(i * k, k)]], rows_v.at[b], sg[b]
                ).wait()
                pltpu.async_copy(
                    rows_v.at[b], out_hbm.at[pl.ds(base + i * k, k)], sw[b])
            return carry

        lax.fori_loop(0, n_pairs, pair, 0)
        for b in range(2):
            pltpu.make_async_copy(
                rows_v.at[b], out_hbm.at[pl.ds(base, k)], sw[b]).wait()

    return gather


def kernel(fmess, bgraph, Wz_w, Wz_b, Wr_w, Ur_w, Ur_b, Wh_w, Wh_b):
    e_total, in_size = fmess.shape
    h_size = Ur_w.shape[0]
    n_nei = bgraph.shape[1]
    depth = 3
    hdtype = jnp.float32
    fdtype = jnp.bfloat16
    ep = e_total // 2
    hp = 2 * h_size

    eye2 = jnp.eye(2, dtype=jnp.float32)
    wzf_t = Wz_w[:, :in_size].T
    wr_t = Wr_w.T
    whf_t = Wh_w[:, :in_size].T
    wzh2 = jnp.kron(eye2, Wz_w[:, in_size:].T)
    whh2 = jnp.kron(eye2, Wh_w[:, in_size:].T)
    ur2 = jnp.kron(eye2, Ur_w.T)
    bz2 = jnp.tile(Wz_b.reshape(1, h_size), (1, 2))
    urb2 = jnp.tile(Ur_b.reshape(1, h_size), (1, 2))
    bh2 = jnp.tile(Wh_b.reshape(1, h_size), (1, 2))
    e2 = e_total // 2
    bg_s = jnp.where(bgraph < e2, 2 * bgraph, 2 * (bgraph - e2) + 1)
    idx = bg_s.T.reshape(n_nei, 2, e2).transpose(0, 2, 1).reshape(-1)

    bp = 1000
    grid_p = ep // bp
    fz, fr, fh, h = pl.pallas_call(
        functools.partial(_precompute_body, block_rows=bp, h_size=h_size),
        grid=(grid_p,),
        in_specs=[
            pl.BlockSpec((bp, in_size), lambda i: (i, 0)),
            pl.BlockSpec((bp, in_size), lambda i: (i + grid_p, 0)),
            pl.BlockSpec((in_size, h_size), lambda i: (0, 0)),
            pl.BlockSpec((in_size, h_size), lambda i: (0, 0)),
            pl.BlockSpec((in_size, h_size), lambda i: (0, 0)),
            pl.BlockSpec((1, hp), lambda i: (0, 0)),
            pl.BlockSpec((1, hp), lambda i: (0, 0)),
        ],
        out_specs=[pl.BlockSpec((bp, hp), lambda i: (i, 0))] * 4,
        out_shape=[jax.ShapeDtypeStruct((ep, hp), fdtype)] * 3
        + [jax.ShapeDtypeStruct((ep, hp), hdtype)],
        compiler_params=pltpu.CompilerParams(
            dimension_semantics=("parallel",)),
    )(fmess, fmess, wzf_t, wr_t, whf_t, bz2, bh2)

    gather = _make_gather(e_total, h_size, n_nei, hdtype)

    bd = 1000
    grid_d = ep // bd

    def make_dense(out_spec, out_shape):
        return pl.pallas_call(
            functools.partial(_dense_body, n_nei=n_nei, block_rows=bd,
                              h_size=h_size),
            grid=(grid_d,),
            in_specs=[
                pl.BlockSpec((n_nei, bd, hp), lambda i: (0, i, 0)),
                pl.BlockSpec((bd, hp), lambda i: (i, 0)),
                pl.BlockSpec((bd, hp), lambda i: (i, 0)),
                pl.BlockSpec((bd, hp), lambda i: (i, 0)),
                pl.BlockSpec((hp, hp), lambda i: (0, 0)),
                pl.BlockSpec((hp, hp), lambda i: (0, 0)),
                pl.BlockSpec((hp, hp), lambda i: (0, 0)),
                pl.BlockSpec((1, hp), lambda i: (0, 0)),
                pl.BlockSpec((1, hp), lambda i: (0, 0)),
                pl.BlockSpec((1, hp), lambda i: (0, 0)),
            ],
            out_specs=out_spec,
            out_shape=out_shape,
            compiler_params=pltpu.CompilerParams(
                dimension_semantics=("parallel",)),
        )

    dense_mid = make_dense(
        pl.BlockSpec((bd, hp), lambda i: (i, 0)),
        jax.ShapeDtypeStruct((ep, hp), hdtype))
    dense_last = make_dense(
        [pl.BlockSpec((bd, h_size), lambda i: (i, 0))] * 2,
        [jax.ShapeDtypeStruct((e2, h_size), jnp.float32)] * 2)

    for step in range(depth - 1):
        hnei = gather(h.reshape(e_total, h_size), idx)
        hnei = hnei.reshape(n_nei, ep, hp)
        if step == depth - 2:
            ha, hb = dense_last(hnei, fz, fr, fh, wzh2, ur2, whh2, bz2, urb2, bh2)
            return jnp.concatenate([ha, hb], axis=0)
        h = dense_mid(hnei, fz, fr, fh, wzh2, ur2, whh2, bz2, urb2, bh2)

    return h


# speedup vs baseline: 1.1276x; 1.1276x over previous
import functools

import jax
import jax.numpy as jnp
from jax import lax
from jax.experimental import pallas as pl
from jax.experimental.pallas import tpu as pltpu
from jax.experimental.pallas import tpu_sc as plsc


def _precompute_body(fm_ref, wz_ref, wr_ref, wh_ref, bz_ref, bh_ref,
                     fz_ref, fr_ref, fh_ref, h1_ref, *, block_rows, h_size):
    fm = fm_ref[...]
    fz = jnp.dot(fm, wz_ref[...], preferred_element_type=jnp.float32)
    fr = jnp.dot(fm, wr_ref[...], preferred_element_type=jnp.float32)
    fh = jnp.dot(fm, wh_ref[...], preferred_element_type=jnp.float32)
    fz_ref[...] = fz.astype(fz_ref.dtype)
    fr_ref[...] = fr.astype(fr_ref.dtype)
    fh_ref[...] = fh.astype(fh_ref.dtype)
    z1 = jax.nn.sigmoid(fz + bz_ref[...])
    p1 = jnp.tanh(fh + bh_ref[...])
    h1 = z1 * p1
    row = lax.broadcasted_iota(jnp.int32, h1.shape, 0) + pl.program_id(0) * block_rows
    lane = lax.broadcasted_iota(jnp.int32, h1.shape, 1)
    h1_ref[...] = jnp.where((row == 0) & (lane < h_size), 0.0, h1).astype(h1_ref.dtype)


def _dense_body(hn_ref, fz_ref, fr_ref, fh_ref, uz_ref, ur_ref, uh_ref,
                bz_ref, urb_ref, bh_ref, out_ref, *, n_nei, block_rows, h_size):
    fr = fr_ref[...].astype(jnp.float32)
    ur = ur_ref[...]
    urb = urb_ref[...]
    sum_h = jnp.zeros_like(fr)
    sum_g = jnp.zeros_like(fr)
    for n in range(n_nei):
        hn = hn_ref[n].astype(jnp.float32)
        r = jax.nn.sigmoid(
            fr + jnp.dot(hn, ur, preferred_element_type=jnp.float32) + urb)
        sum_h = sum_h + hn
        sum_g = sum_g + r * hn
    z = jax.nn.sigmoid(
        fz_ref[...].astype(jnp.float32)
        + jnp.dot(sum_h, uz_ref[...], preferred_element_type=jnp.float32)
        + bz_ref[...])
    pre = jnp.tanh(
        fh_ref[...].astype(jnp.float32)
        + jnp.dot(sum_g, uh_ref[...], preferred_element_type=jnp.float32)
        + bh_ref[...])
    nh = (1.0 - z) * sum_h + z * pre
    row = lax.broadcasted_iota(jnp.int32, nh.shape, 0) + pl.program_id(0) * block_rows
    lane = lax.broadcasted_iota(jnp.int32, nh.shape, 1)
    out_ref[...] = jnp.where((row == 0) & (lane < h_size), 0.0, nh).astype(out_ref.dtype)


def _make_gather(e_total, h_size, n_nei, dtype):
    info = plsc.get_sparse_core_info()
    nc, ns = info.num_cores, info.num_subcores
    nw = nc * ns
    total = e_total * n_nei
    per_w = total // nw
    assert per_w * nw == total and per_w % 8 == 0
    k = 400
    n_chunks = per_w // k
    assert n_chunks * k == per_w and n_chunks % 2 == 0 and k % 8 == 0
    n_pairs = n_chunks // 2

    mesh = plsc.VectorSubcoreMesh(core_axis_name="c", subcore_axis_name="s")

    @functools.partial(
        pl.kernel,
        mesh=mesh,
        out_type=jax.ShapeDtypeStruct((total, h_size), dtype),
        scratch_types=[
            pltpu.VMEM((per_w,), jnp.int32),
            pltpu.VMEM((2, k, h_size), dtype),
            pltpu.SemaphoreType.DMA,
            pltpu.SemaphoreType.DMA,
            pltpu.SemaphoreType.DMA,
            pltpu.SemaphoreType.DMA,
        ],
        compiler_params=pltpu.CompilerParams(use_tc_tiling_on_sc=False),
    )
    def gather(h_hbm, idx_hbm, out_hbm, idx_v, rows_v, sg0, sg1, sw0, sw1):
        wid = lax.axis_index("s") * nc + lax.axis_index("c")
        base = wid * per_w
        sg = (sg0, sg1)
        sw = (sw0, sw1)

        pltpu.sync_copy(idx_hbm.at[pl.ds(base, per_w)], idx_v)

        def pair(g, carry):
            for b in range(2):
                i = g * 2 + b
                @pl.when(g > 0)
                def _():
                    pltpu.make_async_copy(
                        rows_v.at[b], out_hbm.at[pl.ds(base, k)], sw[b]).wait()

                pltpu.async_copy(
                    h_hbm.at[idx_v.at[pl.ds(i * k, k)]], rows_v.at[b], sg[b]
                ).wait()
                pltpu.async_copy(
                    rows_v.at[b], out_hbm.at[pl.ds(base + i * k, k)], sw[b])
            return carry

        lax.fori_loop(0, n_pairs, pair, 0)
        for b in range(2):
            pltpu.make_async_copy(
                rows_v.at[b], out_hbm.at[pl.ds(base, k)], sw[b]).wait()

    return gather


def kernel(fmess, bgraph, Wz_w, Wz_b, Wr_w, Ur_w, Ur_b, Wh_w, Wh_b):
    e_total, in_size = fmess.shape
    h_size = Ur_w.shape[0]
    n_nei = bgraph.shape[1]
    depth = 3
    hdtype = jnp.float32
    fdtype = jnp.bfloat16
    ep = e_total // 2
    hp = 2 * h_size

    eye2 = jnp.eye(2, dtype=jnp.float32)
    wzf2 = jnp.kron(eye2, Wz_w[:, :in_size].T)
    wr2 = jnp.kron(eye2, Wr_w.T)
    whf2 = jnp.kron(eye2, Wh_w[:, :in_size].T)
    wzh2 = jnp.kron(eye2, Wz_w[:, in_size:].T)
    whh2 = jnp.kron(eye2, Wh_w[:, in_size:].T)
    ur2 = jnp.kron(eye2, Ur_w.T)
    bz2 = jnp.tile(Wz_b.reshape(1, h_size), (1, 2))
    urb2 = jnp.tile(Ur_b.reshape(1, h_size), (1, 2))
    bh2 = jnp.tile(Wh_b.reshape(1, h_size), (1, 2))
    idx = bgraph.T.reshape(-1)
    fm2 = fmess.reshape(ep, 2 * in_size)

    bp = 1000
    grid_p = ep // bp
    fz, fr, fh, h = pl.pallas_call(
        functools.partial(_precompute_body, block_rows=bp, h_size=h_size),
        grid=(grid_p,),
        in_specs=[
            pl.BlockSpec((bp, 2 * in_size), lambda i: (i, 0)),
            pl.BlockSpec((2 * in_size, hp), lambda i: (0, 0)),
            pl.BlockSpec((2 * in_size, hp), lambda i: (0, 0)),
            pl.BlockSpec((2 * in_size, hp), lambda i: (0, 0)),
            pl.BlockSpec((1, hp), lambda i: (0, 0)),
            pl.BlockSpec((1, hp), lambda i: (0, 0)),
        ],
        out_specs=[pl.BlockSpec((bp, hp), lambda i: (i, 0))] * 4,
        out_shape=[jax.ShapeDtypeStruct((ep, hp), fdtype)] * 3
        + [jax.ShapeDtypeStruct((ep, hp), hdtype)],
        compiler_params=pltpu.CompilerParams(
            dimension_semantics=("parallel",)),
    )(fm2, wzf2, wr2, whf2, bz2, bh2)

    gather = _make_gather(e_total, h_size, n_nei, hdtype)

    bd = 1000
    grid_d = ep // bd

    def make_dense(out_dtype):
        return pl.pallas_call(
            functools.partial(_dense_body, n_nei=n_nei, block_rows=bd,
                              h_size=h_size),
            grid=(grid_d,),
            in_specs=[
                pl.BlockSpec((n_nei, bd, hp), lambda i: (0, i, 0)),
                pl.BlockSpec((bd, hp), lambda i: (i, 0)),
                pl.BlockSpec((bd, hp), lambda i: (i, 0)),
                pl.BlockSpec((bd, hp), lambda i: (i, 0)),
                pl.BlockSpec((hp, hp), lambda i: (0, 0)),
                pl.BlockSpec((hp, hp), lambda i: (0, 0)),
                pl.BlockSpec((hp, hp), lambda i: (0, 0)),
                pl.BlockSpec((1, hp), lambda i: (0, 0)),
                pl.BlockSpec((1, hp), lambda i: (0, 0)),
                pl.BlockSpec((1, hp), lambda i: (0, 0)),
            ],
            out_specs=pl.BlockSpec((bd, hp), lambda i: (i, 0)),
            out_shape=jax.ShapeDtypeStruct((ep, hp), out_dtype),
            compiler_params=pltpu.CompilerParams(
                dimension_semantics=("parallel",)),
        )

    dense_mid = make_dense(hdtype)
    dense_last = make_dense(jnp.float32)

    for step in range(depth - 1):
        hnei = gather(h.reshape(e_total, h_size), idx)
        hnei = hnei.reshape(n_nei, ep, hp)
        dense = dense_last if step == depth - 2 else dense_mid
        h = dense(hnei, fz, fr, fh, wzh2, ur2, whh2, bz2, urb2, bh2)

    return h.reshape(e_total, h_size)
